# baseline (device time: 7869 ns/iter reference)
import jax
import jax.numpy as jnp
from jax import lax
from jax.experimental import pallas as pl
from jax.experimental.pallas import tpu as pltpu

N_COLS_GLOBAL = 1536
N_CHUNKS = 2


def kernel(x):
    m, n_local = x.shape
    chunk_m = m // N_CHUNKS
    rows = chunk_m // 128

    def body(
        x_hbm, out_ref, x_vmem, send_buf, recv_buf,
        load_sems, send_sems, recv_sems,
    ):
        my_x = lax.axis_index("x")
        my_y = lax.axis_index("y")
        neighbor = (my_x, 1 - my_y)

        barrier_sem = pltpu.get_barrier_semaphore()
        pl.semaphore_signal(
            barrier_sem, inc=1,
            device_id=neighbor, device_id_type=pl.DeviceIdType.MESH,
        )

        loads = []
        for c in range(N_CHUNKS):
            cp = pltpu.make_async_copy(
                x_hbm.at[pl.ds(c * chunk_m, chunk_m), :],
                x_vmem.at[c],
                load_sems.at[c],
            )
            cp.start()
            loads.append(cp)

        rdmas = []
        for c in range(N_CHUNKS):
            loads[c].wait()
            partial = jnp.sum(
                x_vmem[c].astype(jnp.float32), axis=1, keepdims=True
            )
            send_buf[pl.ds(c * rows, rows)] = partial.reshape(rows, 128)

            if c == 0:
                pl.semaphore_wait(barrier_sem, 1)

            rdma = pltpu.make_async_remote_copy(
                src_ref=send_buf.at[pl.ds(c * rows, rows)],
                dst_ref=recv_buf.at[pl.ds(c * rows, rows)],
                send_sem=send_sems.at[c],
                recv_sem=recv_sems.at[c],
                device_id=neighbor,
                device_id_type=pl.DeviceIdType.MESH,
            )
            rdma.start()
            rdmas.append(rdma)

        for rdma in rdmas:
            rdma.wait_send()
            rdma.wait_recv()

        total = (send_buf[:, :] + recv_buf[:, :]) * (1.0 / N_COLS_GLOBAL)
        tcol = total.T
        for a in range(m // 128):
            out_ref[pl.ds(a * 128, 128), :] = tcol[:, a : a + 1]

    return pl.pallas_call(
        body,
        out_shape=jax.ShapeDtypeStruct((m, 1), jnp.float32),
        in_specs=[pl.BlockSpec(memory_space=pl.ANY)],
        out_specs=pl.BlockSpec(memory_space=pltpu.VMEM),
        scratch_shapes=[
            pltpu.VMEM((N_CHUNKS, chunk_m, n_local), x.dtype),
            pltpu.VMEM((m // 128, 128), jnp.float32),
            pltpu.VMEM((m // 128, 128), jnp.float32),
            pltpu.SemaphoreType.DMA((N_CHUNKS,)),
            pltpu.SemaphoreType.DMA((N_CHUNKS,)),
            pltpu.SemaphoreType.DMA((N_CHUNKS,)),
        ],
        compiler_params=pltpu.CompilerParams(collective_id=0),
    )(x)


# device time: 7473 ns/iter; 1.0530x vs baseline; 1.0530x over previous
import jax
import jax.numpy as jnp
from jax import lax
from jax.experimental import pallas as pl
from jax.experimental.pallas import tpu as pltpu

N_COLS_GLOBAL = 1536
N_CHUNKS = 2


def kernel(x):
    m, _ = x.shape
    chunk_m = m // N_CHUNKS
    rows = chunk_m // 128

    def body(x_ref, out_ref, send_buf, recv_buf, send_sems, recv_sems):
        my_x = lax.axis_index("x")
        my_y = lax.axis_index("y")
        neighbor = (my_x, 1 - my_y)

        barrier_sem = pltpu.get_barrier_semaphore()
        pl.semaphore_signal(
            barrier_sem, inc=1,
            device_id=neighbor, device_id_type=pl.DeviceIdType.MESH,
        )

        rdmas = []
        for c in range(N_CHUNKS):
            partial = jnp.sum(
                x_ref[pl.ds(c * chunk_m, chunk_m), :].astype(jnp.float32),
                axis=1,
                keepdims=True,
            ) * (1.0 / N_COLS_GLOBAL)
            send_buf[pl.ds(c * rows, rows)] = partial.reshape(rows, 128)

            if c == 0:
                pl.semaphore_wait(barrier_sem, 1)

            rdma = pltpu.make_async_remote_copy(
                src_ref=send_buf.at[pl.ds(c * rows, rows)],
                dst_ref=recv_buf.at[pl.ds(c * rows, rows)],
                send_sem=send_sems.at[c],
                recv_sem=recv_sems.at[c],
                device_id=neighbor,
                device_id_type=pl.DeviceIdType.MESH,
            )
            rdma.start()
            rdmas.append(rdma)

        for c, rdma in enumerate(rdmas):
            rdma.wait_recv()
            total = (
                send_buf[pl.ds(c * rows, rows)] + recv_buf[pl.ds(c * rows, rows)]
            )
            tcol = total.T
            for a in range(rows):
                out_ref[pl.ds((c * rows + a) * 128, 128), :] = tcol[:, a : a + 1]

        for rdma in rdmas:
            rdma.wait_send()

    return pl.pallas_call(
        body,
        out_shape=jax.ShapeDtypeStruct((m, 1), jnp.float32),
        in_specs=[pl.BlockSpec(memory_space=pltpu.VMEM)],
        out_specs=pl.BlockSpec(memory_space=pltpu.VMEM),
        scratch_shapes=[
            pltpu.VMEM((m // 128, 128), jnp.float32),
            pltpu.VMEM((m // 128, 128), jnp.float32),
            pltpu.SemaphoreType.DMA((N_CHUNKS,)),
            pltpu.SemaphoreType.DMA((N_CHUNKS,)),
        ],
        compiler_params=pltpu.CompilerParams(collective_id=0),
    )(x)
